# Initial kernel scaffold; baseline (speedup 1.0000x reference)
import jax
import jax.numpy as jnp
from jax.experimental import pallas as pl


def kernel(x, edge_index, Wq1, bq1, Wk1, bk1, Wv1, bv1, Ws1, bs1, Wq2, bq2, Wk2, bk2, Wv2, bv2, Ws2, bs2, Wo, bo):
    raise NotImplementedError("write your pallas kernel here")


# speedup vs baseline: 18.9485x; 18.9485x over previous
import functools

import jax
import jax.numpy as jnp
from jax import lax
from jax.experimental import pallas as pl
from jax.experimental.pallas import tpu as pltpu
from jax.experimental.pallas import tpu_sc as plsc

N = 10000
E = 320000
D = 128
C = 11
CP = 16
W = 32
GT = 80
GROUP = 128
EP = W * GT * GROUP
ROWS_PER_TILE = N // 16
INV_SQRT_C = 1.0 / (C ** 0.5)


_BLK = 2000


def _project_body(x_ref, w_ref, b_ref, q_ref, k_ref, v_ref, s_ref):
    h = jnp.dot(x_ref[...], w_ref[...], preferred_element_type=jnp.float32)
    h = h + b_ref[...]
    q_ref[...] = h[:, 0:16]
    k_ref[...] = h[:, 16:32]
    v_ref[...] = h[:, 32:48]
    s_ref[...] = h[:, 48:64]


def _tc_project(x, wcat, bcat, in_dim):
    blk = _BLK
    grid = N // blk
    out = jax.ShapeDtypeStruct((N, CP), jnp.float32)
    return pl.pallas_call(
        _project_body,
        grid=(grid,),
        in_specs=[
            pl.BlockSpec((blk, in_dim), lambda i: (i, 0)),
            pl.BlockSpec((in_dim, 4 * CP), lambda i: (0, 0)),
            pl.BlockSpec((1, 4 * CP), lambda i: (0, 0)),
        ],
        out_specs=[pl.BlockSpec((blk, CP), lambda i: (i, 0))] * 4,
        out_shape=[out, out, out, out],
    )(x, wcat, bcat)


def _merge_body(acc_ref, skip_ref, h_ref):
    a = acc_ref[0] + acc_ref[1]
    wsum = a[:, 11:12]
    h = a / (wsum + 1e-16) + skip_ref[...]
    h = jnp.maximum(h, 0.0)
    col = lax.broadcasted_iota(jnp.int32, h.shape, 1)
    h_ref[...] = jnp.where(col < C, h, 0.0)


def _tc_merge(acc, skip):
    blk = _BLK
    return pl.pallas_call(
        _merge_body,
        grid=(N // blk,),
        in_specs=[
            pl.BlockSpec((2, blk, CP), lambda i: (0, i, 0)),
            pl.BlockSpec((blk, CP), lambda i: (i, 0)),
        ],
        out_specs=pl.BlockSpec((blk, CP), lambda i: (i, 0)),
        out_shape=jax.ShapeDtypeStruct((N, CP), jnp.float32),
    )(acc, skip)


def _final_body(acc_ref, skip_ref, wo_ref, bo_ref, y_ref):
    a = acc_ref[0] + acc_ref[1]
    wsum = a[:, 11:12]
    h = a / (wsum + 1e-16) + skip_ref[...]
    h = jnp.maximum(h, 0.0)
    col = lax.broadcasted_iota(jnp.int32, h.shape, 1)
    h = jnp.where(col < C, h, 0.0)
    z = jnp.dot(h, wo_ref[...], preferred_element_type=jnp.float32)
    z = z + bo_ref[...]
    y_ref[...] = 1.0 / (1.0 + jnp.exp(-z))


def _tc_final(acc, skip, wo_pad, bo):
    blk = _BLK
    return pl.pallas_call(
        _final_body,
        grid=(N // blk,),
        in_specs=[
            pl.BlockSpec((2, blk, CP), lambda i: (0, i, 0)),
            pl.BlockSpec((blk, CP), lambda i: (i, 0)),
            pl.BlockSpec((CP, D), lambda i: (0, 0)),
            pl.BlockSpec((1, D), lambda i: (0, 0)),
        ],
        out_specs=pl.BlockSpec((blk, D), lambda i: (i, 0)),
        out_shape=jax.ShapeDtypeStruct((N, D), jnp.float32),
    )(acc, skip, wo_pad, bo)


@functools.lru_cache(maxsize=1)
def _sc_mesh():
    return plsc.VectorSubcoreMesh(
        core_axis_name="c", subcore_axis_name="s", num_cores=2, num_subcores=16)


def _sc_alpha_body(q_hbm, k_hbm, src_hbm, dst_hbm,
                   alpha_out, mx_out,
                   src_v, dst_v, qrows, krows, alpha_v, mx_v,
                   sem_q, sem_k):
    c = lax.axis_index("c")
    s = lax.axis_index("s")
    wid = c * 16 + s
    base_g = wid * GT
    pltpu.sync_copy(src_hbm.at[pl.ds(base_g, GT)], src_v)
    pltpu.sync_copy(dst_hbm.at[pl.ds(base_g, GT)], dst_v)

    lane = lax.iota(jnp.int32, 16)

    def group(g, mx):
        cq = pltpu.async_copy(q_hbm.at[dst_v.at[g]], qrows, sem_q)
        ck = pltpu.async_copy(k_hbm.at[src_v.at[g]], krows, sem_k)
        cq.wait()
        ck.wait()
        for sub in range(8):
            idx = lane + (sub * 16)
            acc = jnp.zeros((16,), jnp.float32)
            for ch in range(C):
                chv = jnp.full((16,), ch, jnp.int32)
                qc = plsc.load_gather(qrows, [idx, chv])
                kc = plsc.load_gather(krows, [idx, chv])
                acc = acc + qc * kc
            acc = acc * INV_SQRT_C
            alpha_v[pl.ds(sub * 16, 16)] = acc
            mx = jnp.maximum(mx, acc)
        pltpu.sync_copy(alpha_v, alpha_out.at[base_g + g])
        return mx

    mx = lax.fori_loop(0, GT, group, jnp.full((16,), -1e30, jnp.float32))
    mx_v[...] = mx
    pltpu.sync_copy(mx_v, mx_out.at[wid])


@functools.lru_cache(maxsize=1)
def _sc_alpha():
    return pl.kernel(
        _sc_alpha_body,
        out_type=[
            jax.ShapeDtypeStruct((W * GT, GROUP), jnp.float32),
            jax.ShapeDtypeStruct((W, 16), jnp.float32),
        ],
        mesh=_sc_mesh(),
        scratch_types=[
            pltpu.VMEM((GT, GROUP), jnp.int32),
            pltpu.VMEM((GT, GROUP), jnp.int32),
            pltpu.VMEM((GROUP, CP), jnp.float32),
            pltpu.VMEM((GROUP, CP), jnp.float32),
            pltpu.VMEM((GROUP,), jnp.float32),
            pltpu.VMEM((16,), jnp.float32),
            pltpu.SemaphoreType.DMA,
            pltpu.SemaphoreType.DMA,
        ],
        compiler_params=pltpu.CompilerParams(use_tc_tiling_on_sc=False, needs_layout_passes=False),
    )


def _sc_message_body(v_hbm, alpha_hbm, mx_hbm, src_hbm, dst_hbm,
                     acc_out,
                     src_v, dst_v, alpha_vt, vrows, msg, mxv, tmp,
                     acc_spmem, sem_v):
    c = lax.axis_index("c")
    s = lax.axis_index("s")
    wid = c * 16 + s
    base_g = wid * GT

    pltpu.sync_copy(src_hbm.at[pl.ds(base_g, GT)], src_v)
    pltpu.sync_copy(dst_hbm.at[pl.ds(base_g, GT)], dst_v)
    pltpu.sync_copy(alpha_hbm.at[pl.ds(base_g, GT)], alpha_vt)
    pltpu.sync_copy(mx_hbm, mxv)

    m = jnp.full((16,), -1e30, jnp.float32)
    for i in range(W):
        m = jnp.maximum(m, mxv[i, :])
    gmax = jnp.max(m)

    lane = lax.iota(jnp.int32, 16)
    zero16 = jnp.zeros((16,), jnp.float32)

    def zrow(i, _):
        tmp[i, :] = zero16
        return 0
    lax.fori_loop(0, ROWS_PER_TILE, zrow, 0)

    def zmsg(i, _):
        msg[i, :] = zero16
        return 0
    lax.fori_loop(0, GROUP, zmsg, 0)

    pltpu.sync_copy(tmp, acc_spmem.at[pl.ds(s * ROWS_PER_TILE, ROWS_PER_TILE)])
    plsc.subcore_barrier()

    def group(g, _):
        cv = pltpu.async_copy(v_hbm.at[src_v.at[g]], vrows, sem_v)
        cv.wait()
        for sub in range(8):
            idx = lane + (sub * 16)
            a = alpha_vt[g, pl.ds(sub * 16, 16)]
            ae = jnp.exp(a - gmax)
            gid = (base_g + g) * GROUP + sub * 16 + lane
            ae = jnp.where(gid < E, ae, 0.0)
            for ch in range(C):
                chv = jnp.full((16,), ch, jnp.int32)
                vc = plsc.load_gather(vrows, [idx, chv])
                plsc.store_scatter(msg, [idx, chv], vc * ae)
            plsc.store_scatter(msg, [idx, jnp.full((16,), C, jnp.int32)], ae)
        pltpu.sync_copy(msg, acc_spmem.at[dst_v.at[g]], add=True)
        return 0

    lax.fori_loop(0, GT, group, 0)
    plsc.subcore_barrier()

    pltpu.sync_copy(acc_spmem.at[pl.ds(s * ROWS_PER_TILE, ROWS_PER_TILE)], tmp)
    pltpu.sync_copy(tmp, acc_out.at[c, pl.ds(s * ROWS_PER_TILE, ROWS_PER_TILE)])


@functools.lru_cache(maxsize=1)
def _sc_message():
    return pl.kernel(
        _sc_message_body,
        out_type=jax.ShapeDtypeStruct((2, N, CP), jnp.float32),
        mesh=_sc_mesh(),
        scratch_types=[
            pltpu.VMEM((GT, GROUP), jnp.int32),
            pltpu.VMEM((GT, GROUP), jnp.int32),
            pltpu.VMEM((GT, GROUP), jnp.float32),
            pltpu.VMEM((GROUP, CP), jnp.float32),
            pltpu.VMEM((GROUP, CP), jnp.float32),
            pltpu.VMEM((W, 16), jnp.float32),
            pltpu.VMEM((ROWS_PER_TILE, CP), jnp.float32),
            pltpu.VMEM_SHARED((N, CP), jnp.float32),
            pltpu.SemaphoreType.DMA,
        ],
        compiler_params=pltpu.CompilerParams(use_tc_tiling_on_sc=False, needs_layout_passes=False),
    )


def _pack_weights(wq, bq, wk, bk, wv, bv, ws, bs, in_dim):
    wcat = jnp.zeros((in_dim, 4 * CP), jnp.float32)
    bcat = jnp.zeros((1, 4 * CP), jnp.float32)
    for slot, (w_, b_) in enumerate(((wq, bq), (wk, bk), (wv, bv), (ws, bs))):
        wcat = wcat.at[: w_.shape[0], slot * CP: slot * CP + C].set(w_)
        bcat = bcat.at[0, slot * CP: slot * CP + C].set(b_)
    return wcat, bcat


def kernel(x, edge_index, Wq1, bq1, Wk1, bk1, Wv1, bv1, Ws1, bs1,
           Wq2, bq2, Wk2, bk2, Wv2, bv2, Ws2, bs2, Wo, bo):
    src = jnp.pad(edge_index[0], (0, EP - E)).reshape(W * GT, GROUP)
    dst = jnp.pad(edge_index[1], (0, EP - E)).reshape(W * GT, GROUP)

    wcat1, bcat1 = _pack_weights(Wq1, bq1, Wk1, bk1, Wv1, bv1, Ws1, bs1, D)
    wcat2, bcat2 = _pack_weights(Wq2, bq2, Wk2, bk2, Wv2, bv2, Ws2, bs2, CP)
    wo_pad = jnp.zeros((CP, D), jnp.float32).at[:C, :].set(Wo)
    bo2 = bo.reshape(1, D)

    sc_alpha = _sc_alpha()
    sc_message = _sc_message()

    q1, k1, v1, s1 = _tc_project(x, wcat1, bcat1, D)
    alpha1, mx1 = sc_alpha(q1, k1, src, dst)
    acc1 = sc_message(v1, alpha1, mx1, src, dst)
    h1 = _tc_merge(acc1, s1)

    q2, k2, v2, s2 = _tc_project(h1, wcat2, bcat2, CP)
    alpha2, mx2 = sc_alpha(q2, k2, src, dst)
    acc2 = sc_message(v2, alpha2, mx2, src, dst)

    return _tc_final(acc2, s2, wo_pad, bo2)


# speedup vs baseline: 30.5147x; 1.6104x over previous
import functools

import jax
import jax.numpy as jnp
from jax import lax
from jax.experimental import pallas as pl
from jax.experimental.pallas import tpu as pltpu
from jax.experimental.pallas import tpu_sc as plsc

N = 10000
E = 320000
D = 128
C = 11
CP = 16
W = 32
GT = 80
GROUP = 128
EP = W * GT * GROUP
ROWS_PER_TILE = N // 16
INV_SQRT_C = 1.0 / (C ** 0.5)


_BLK = 2000


def _project_body(x_ref, w_ref, b_ref, q_ref, k_ref, v_ref, s_ref):
    h = jnp.dot(x_ref[...], w_ref[...], preferred_element_type=jnp.float32)
    h = h + b_ref[...]
    q_ref[...] = h[:, 0:16]
    k_ref[...] = h[:, 16:32]
    v_ref[...] = h[:, 32:48]
    s_ref[...] = h[:, 48:64]


def _tc_project(x, wcat, bcat, in_dim):
    blk = _BLK
    grid = N // blk
    out = jax.ShapeDtypeStruct((N, CP), jnp.float32)
    return pl.pallas_call(
        _project_body,
        grid=(grid,),
        in_specs=[
            pl.BlockSpec((blk, in_dim), lambda i: (i, 0)),
            pl.BlockSpec((in_dim, 4 * CP), lambda i: (0, 0)),
            pl.BlockSpec((1, 4 * CP), lambda i: (0, 0)),
        ],
        out_specs=[pl.BlockSpec((blk, CP), lambda i: (i, 0))] * 4,
        out_shape=[out, out, out, out],
    )(x, wcat, bcat)


def _merge_body(acc_ref, skip_ref, h_ref):
    a = acc_ref[0] + acc_ref[1]
    wsum = a[:, 11:12]
    h = a / (wsum + 1e-16) + skip_ref[...]
    h = jnp.maximum(h, 0.0)
    col = lax.broadcasted_iota(jnp.int32, h.shape, 1)
    h_ref[...] = jnp.where(col < C, h, 0.0)


def _tc_merge(acc, skip):
    blk = _BLK
    return pl.pallas_call(
        _merge_body,
        grid=(N // blk,),
        in_specs=[
            pl.BlockSpec((2, blk, CP), lambda i: (0, i, 0)),
            pl.BlockSpec((blk, CP), lambda i: (i, 0)),
        ],
        out_specs=pl.BlockSpec((blk, CP), lambda i: (i, 0)),
        out_shape=jax.ShapeDtypeStruct((N, CP), jnp.float32),
    )(acc, skip)


def _final_body(acc_ref, skip_ref, wo_ref, bo_ref, y_ref):
    a = acc_ref[0] + acc_ref[1]
    wsum = a[:, 11:12]
    h = a / (wsum + 1e-16) + skip_ref[...]
    h = jnp.maximum(h, 0.0)
    col = lax.broadcasted_iota(jnp.int32, h.shape, 1)
    h = jnp.where(col < C, h, 0.0)
    z = jnp.dot(h, wo_ref[...], preferred_element_type=jnp.float32)
    z = z + bo_ref[...]
    y_ref[...] = 1.0 / (1.0 + jnp.exp(-z))


def _tc_final(acc, skip, wo_pad, bo):
    blk = _BLK
    return pl.pallas_call(
        _final_body,
        grid=(N // blk,),
        in_specs=[
            pl.BlockSpec((2, blk, CP), lambda i: (0, i, 0)),
            pl.BlockSpec((blk, CP), lambda i: (i, 0)),
            pl.BlockSpec((CP, D), lambda i: (0, 0)),
            pl.BlockSpec((1, D), lambda i: (0, 0)),
        ],
        out_specs=pl.BlockSpec((blk, D), lambda i: (i, 0)),
        out_shape=jax.ShapeDtypeStruct((N, D), jnp.float32),
    )(acc, skip, wo_pad, bo)


@functools.lru_cache(maxsize=1)
def _sc_mesh():
    return plsc.VectorSubcoreMesh(
        core_axis_name="c", subcore_axis_name="s", num_cores=2, num_subcores=16)


def _sc_alpha_body(q_hbm, k_hbm, src_hbm, dst_hbm,
                   alpha_out, mx_out,
                   src_v, dst_v, qr0, kr0, qr1, kr1, alpha_v, mx_v,
                   sem_q0, sem_k0, sem_q1, sem_k1):
    c = lax.axis_index("c")
    s = lax.axis_index("s")
    wid = c * 16 + s
    base_g = wid * GT
    pltpu.sync_copy(src_hbm.at[pl.ds(base_g, GT)], src_v)
    pltpu.sync_copy(dst_hbm.at[pl.ds(base_g, GT)], dst_v)

    lane = lax.iota(jnp.int32, 16)

    def start(g, qr, kr, sq, sk):
        pltpu.make_async_copy(q_hbm.at[dst_v.at[g]], qr, sq).start()
        pltpu.make_async_copy(k_hbm.at[src_v.at[g]], kr, sk).start()

    def wait(g, qr, kr, sq, sk):
        pltpu.make_async_copy(q_hbm.at[dst_v.at[g]], qr, sq).wait()
        pltpu.make_async_copy(k_hbm.at[src_v.at[g]], kr, sk).wait()

    def compute(g, qr, kr, mx):
        for sub in range(8):
            idx = lane + (sub * 16)
            acc = jnp.zeros((16,), jnp.float32)
            for ch in range(C):
                chv = jnp.full((16,), ch, jnp.int32)
                qc = plsc.load_gather(qr, [idx, chv])
                kc = plsc.load_gather(kr, [idx, chv])
                acc = acc + qc * kc
            acc = acc * INV_SQRT_C
            alpha_v[pl.ds(sub * 16, 16)] = acc
            mx = jnp.maximum(mx, acc)
        pltpu.sync_copy(alpha_v, alpha_out.at[base_g + g])
        return mx

    start(0, qr0, kr0, sem_q0, sem_k0)

    def pair(it, mx):
        g0 = 2 * it
        g1 = g0 + 1
        start(g1, qr1, kr1, sem_q1, sem_k1)
        wait(g0, qr0, kr0, sem_q0, sem_k0)
        mx = compute(g0, qr0, kr0, mx)

        @pl.when(it < GT // 2 - 1)
        def _():
            start(g0 + 2, qr0, kr0, sem_q0, sem_k0)

        wait(g1, qr1, kr1, sem_q1, sem_k1)
        mx = compute(g1, qr1, kr1, mx)
        return mx

    mx = lax.fori_loop(0, GT // 2, pair, jnp.full((16,), -1e30, jnp.float32))
    mx_v[...] = mx
    pltpu.sync_copy(mx_v, mx_out.at[wid])


@functools.lru_cache(maxsize=1)
def _sc_alpha():
    return pl.kernel(
        _sc_alpha_body,
        out_type=[
            jax.ShapeDtypeStruct((W * GT, GROUP), jnp.float32),
            jax.ShapeDtypeStruct((W, 16), jnp.float32),
        ],
        mesh=_sc_mesh(),
        scratch_types=[
            pltpu.VMEM((GT, GROUP), jnp.int32),
            pltpu.VMEM((GT, GROUP), jnp.int32),
            pltpu.VMEM((GROUP, CP), jnp.float32),
            pltpu.VMEM((GROUP, CP), jnp.float32),
            pltpu.VMEM((GROUP, CP), jnp.float32),
            pltpu.VMEM((GROUP, CP), jnp.float32),
            pltpu.VMEM((GROUP,), jnp.float32),
            pltpu.VMEM((16,), jnp.float32),
            pltpu.SemaphoreType.DMA,
            pltpu.SemaphoreType.DMA,
            pltpu.SemaphoreType.DMA,
            pltpu.SemaphoreType.DMA,
        ],
        compiler_params=pltpu.CompilerParams(use_tc_tiling_on_sc=False, needs_layout_passes=False),
    )


def _sc_message_body(v_hbm, alpha_hbm, mx_hbm, src_hbm, dst_hbm,
                     acc_out,
                     src_v, dst_v, alpha_vt, vr0, vr1, msg0, msg1, mxv, tmp,
                     acc_spmem, sem_v0, sem_v1, sem_s0, sem_s1):
    c = lax.axis_index("c")
    s = lax.axis_index("s")
    wid = c * 16 + s
    base_g = wid * GT

    pltpu.sync_copy(src_hbm.at[pl.ds(base_g, GT)], src_v)
    pltpu.sync_copy(dst_hbm.at[pl.ds(base_g, GT)], dst_v)
    pltpu.sync_copy(alpha_hbm.at[pl.ds(base_g, GT)], alpha_vt)
    pltpu.sync_copy(mx_hbm, mxv)

    m = jnp.full((16,), -1e30, jnp.float32)
    for i in range(W):
        m = jnp.maximum(m, mxv[i, :])
    gmax = jnp.max(m)

    lane = lax.iota(jnp.int32, 16)
    zero16 = jnp.zeros((16,), jnp.float32)

    def zrow(i, _):
        tmp[i, :] = zero16
        return 0
    lax.fori_loop(0, ROWS_PER_TILE, zrow, 0)

    def zmsg(i, _):
        msg0[i, :] = zero16
        msg1[i, :] = zero16
        return 0
    lax.fori_loop(0, GROUP, zmsg, 0)

    pltpu.sync_copy(tmp, acc_spmem.at[pl.ds(s * ROWS_PER_TILE, ROWS_PER_TILE)])
    plsc.subcore_barrier()

    def start_v(g, vr, sem):
        pltpu.make_async_copy(v_hbm.at[src_v.at[g]], vr, sem).start()

    def wait_v(g, vr, sem):
        pltpu.make_async_copy(v_hbm.at[src_v.at[g]], vr, sem).wait()

    def wait_scatter(msg, sem):
        pltpu.make_async_copy(msg, acc_spmem.at[dst_v.at[0]], sem).wait()

    def compute(g, vr, msg):
        for sub in range(8):
            idx = lane + (sub * 16)
            a = alpha_vt[g, pl.ds(sub * 16, 16)]
            ae = jnp.exp(a - gmax)
            gid = (base_g + g) * GROUP + sub * 16 + lane
            ae = jnp.where(gid < E, ae, 0.0)
            for ch in range(C):
                chv = jnp.full((16,), ch, jnp.int32)
                vc = plsc.load_gather(vr, [idx, chv])
                plsc.store_scatter(msg, [idx, chv], vc * ae)
            plsc.store_scatter(msg, [idx, jnp.full((16,), C, jnp.int32)], ae)
        pltpu.async_copy(msg, acc_spmem.at[dst_v.at[g]], (sem_s0 if msg is msg0 else sem_s1), add=True)

    start_v(0, vr0, sem_v0)

    def pair(it, _):
        g0 = 2 * it
        g1 = g0 + 1
        start_v(g1, vr1, sem_v1)
        wait_v(g0, vr0, sem_v0)

        @pl.when(it > 0)
        def _():
            wait_scatter(msg0, sem_s0)

        compute(g0, vr0, msg0)

        @pl.when(it < GT // 2 - 1)
        def _():
            start_v(g0 + 2, vr0, sem_v0)

        wait_v(g1, vr1, sem_v1)

        @pl.when(it > 0)
        def _():
            wait_scatter(msg1, sem_s1)

        compute(g1, vr1, msg1)
        return 0

    lax.fori_loop(0, GT // 2, pair, 0)
    wait_scatter(msg0, sem_s0)
    wait_scatter(msg1, sem_s1)
    plsc.subcore_barrier()

    pltpu.sync_copy(acc_spmem.at[pl.ds(s * ROWS_PER_TILE, ROWS_PER_TILE)], tmp)
    pltpu.sync_copy(tmp, acc_out.at[c, pl.ds(s * ROWS_PER_TILE, ROWS_PER_TILE)])


@functools.lru_cache(maxsize=1)
def _sc_message():
    return pl.kernel(
        _sc_message_body,
        out_type=jax.ShapeDtypeStruct((2, N, CP), jnp.float32),
        mesh=_sc_mesh(),
        scratch_types=[
            pltpu.VMEM((GT, GROUP), jnp.int32),
            pltpu.VMEM((GT, GROUP), jnp.int32),
            pltpu.VMEM((GT, GROUP), jnp.float32),
            pltpu.VMEM((GROUP, CP), jnp.float32),
            pltpu.VMEM((GROUP, CP), jnp.float32),
            pltpu.VMEM((GROUP, CP), jnp.float32),
            pltpu.VMEM((GROUP, CP), jnp.float32),
            pltpu.VMEM((W, 16), jnp.float32),
            pltpu.VMEM((ROWS_PER_TILE, CP), jnp.float32),
            pltpu.VMEM_SHARED((N, CP), jnp.float32),
            pltpu.SemaphoreType.DMA,
            pltpu.SemaphoreType.DMA,
            pltpu.SemaphoreType.DMA,
            pltpu.SemaphoreType.DMA,
        ],
        compiler_params=pltpu.CompilerParams(use_tc_tiling_on_sc=False, needs_layout_passes=False),
    )


def _pack_weights(wq, bq, wk, bk, wv, bv, ws, bs, in_dim):
    wcat = jnp.zeros((in_dim, 4 * CP), jnp.float32)
    bcat = jnp.zeros((1, 4 * CP), jnp.float32)
    for slot, (w_, b_) in enumerate(((wq, bq), (wk, bk), (wv, bv), (ws, bs))):
        wcat = wcat.at[: w_.shape[0], slot * CP: slot * CP + C].set(w_)
        bcat = bcat.at[0, slot * CP: slot * CP + C].set(b_)
    return wcat, bcat


def kernel(x, edge_index, Wq1, bq1, Wk1, bk1, Wv1, bv1, Ws1, bs1,
           Wq2, bq2, Wk2, bk2, Wv2, bv2, Ws2, bs2, Wo, bo):
    src = jnp.pad(edge_index[0], (0, EP - E)).reshape(W * GT, GROUP)
    dst = jnp.pad(edge_index[1], (0, EP - E)).reshape(W * GT, GROUP)

    wcat1, bcat1 = _pack_weights(Wq1, bq1, Wk1, bk1, Wv1, bv1, Ws1, bs1, D)
    wcat2, bcat2 = _pack_weights(Wq2, bq2, Wk2, bk2, Wv2, bv2, Ws2, bs2, CP)
    wo_pad = jnp.zeros((CP, D), jnp.float32).at[:C, :].set(Wo)
    bo2 = bo.reshape(1, D)

    sc_alpha = _sc_alpha()
    sc_message = _sc_message()

    q1, k1, v1, s1 = _tc_project(x, wcat1, bcat1, D)
    alpha1, mx1 = sc_alpha(q1, k1, src, dst)
    acc1 = sc_message(v1, alpha1, mx1, src, dst)
    h1 = _tc_merge(acc1, s1)

    q2, k2, v2, s2 = _tc_project(h1, wcat2, bcat2, CP)
    alpha2, mx2 = sc_alpha(q2, k2, src, dst)
    acc2 = sc_message(v2, alpha2, mx2, src, dst)

    return _tc_final(acc2, s2, wo_pad, bo2)


# speedup vs baseline: 33.1739x; 1.0871x over previous
import functools

import jax
import jax.numpy as jnp
from jax import lax
from jax.experimental import pallas as pl
from jax.experimental.pallas import tpu as pltpu
from jax.experimental.pallas import tpu_sc as plsc

N = 10000
E = 320000
D = 128
C = 11
CP = 16
W = 32
GT = 80
GROUP = 128
EP = W * GT * GROUP
ROWS_PER_TILE = N // 16
INV_SQRT_C = 1.0 / (C ** 0.5)


_BLK = 2000


def _project_body(x_ref, w_ref, b_ref, q_ref, k_ref, v_ref, s_ref):
    h = jnp.dot(x_ref[...], w_ref[...], preferred_element_type=jnp.float32)
    h = h + b_ref[...]
    q_ref[...] = h[:, 0:16]
    k_ref[...] = h[:, 16:32]
    v_ref[...] = h[:, 32:48]
    s_ref[...] = h[:, 48:64]


def _tc_project(x, wcat, bcat, in_dim):
    blk = _BLK
    grid = N // blk
    out = jax.ShapeDtypeStruct((N, CP), jnp.float32)
    return pl.pallas_call(
        _project_body,
        grid=(grid,),
        in_specs=[
            pl.BlockSpec((blk, in_dim), lambda i: (i, 0)),
            pl.BlockSpec((in_dim, 4 * CP), lambda i: (0, 0)),
            pl.BlockSpec((1, 4 * CP), lambda i: (0, 0)),
        ],
        out_specs=[pl.BlockSpec((blk, CP), lambda i: (i, 0))] * 4,
        out_shape=[out, out, out, out],
    )(x, wcat, bcat)


def _merge_body(acc_ref, skip_ref, h_ref):
    a = acc_ref[0] + acc_ref[1]
    wsum = a[:, 11:12]
    h = a / (wsum + 1e-16) + skip_ref[...]
    h = jnp.maximum(h, 0.0)
    col = lax.broadcasted_iota(jnp.int32, h.shape, 1)
    h_ref[...] = jnp.where(col < C, h, 0.0)


def _tc_merge(acc, skip):
    blk = _BLK
    return pl.pallas_call(
        _merge_body,
        grid=(N // blk,),
        in_specs=[
            pl.BlockSpec((2, blk, CP), lambda i: (0, i, 0)),
            pl.BlockSpec((blk, CP), lambda i: (i, 0)),
        ],
        out_specs=pl.BlockSpec((blk, CP), lambda i: (i, 0)),
        out_shape=jax.ShapeDtypeStruct((N, CP), jnp.float32),
    )(acc, skip)


def _final_body(acc_ref, skip_ref, wo_ref, bo_ref, y_ref):
    a = acc_ref[0] + acc_ref[1]
    wsum = a[:, 11:12]
    h = a / (wsum + 1e-16) + skip_ref[...]
    h = jnp.maximum(h, 0.0)
    col = lax.broadcasted_iota(jnp.int32, h.shape, 1)
    h = jnp.where(col < C, h, 0.0)
    z = jnp.dot(h, wo_ref[...], preferred_element_type=jnp.float32)
    z = z + bo_ref[...]
    y_ref[...] = 1.0 / (1.0 + jnp.exp(-z))


def _tc_final(acc, skip, wo_pad, bo):
    blk = _BLK
    return pl.pallas_call(
        _final_body,
        grid=(N // blk,),
        in_specs=[
            pl.BlockSpec((2, blk, CP), lambda i: (0, i, 0)),
            pl.BlockSpec((blk, CP), lambda i: (i, 0)),
            pl.BlockSpec((CP, D), lambda i: (0, 0)),
            pl.BlockSpec((1, D), lambda i: (0, 0)),
        ],
        out_specs=pl.BlockSpec((blk, D), lambda i: (i, 0)),
        out_shape=jax.ShapeDtypeStruct((N, D), jnp.float32),
    )(acc, skip, wo_pad, bo)


@functools.lru_cache(maxsize=1)
def _sc_mesh():
    return plsc.VectorSubcoreMesh(
        core_axis_name="c", subcore_axis_name="s", num_cores=2, num_subcores=16)


_NBUF = 4


def _sc_alpha_body(q_hbm, k_hbm, src_hbm, dst_hbm,
                   alpha_out, mx_out,
                   src_v, dst_v, qr_all, kr_all, alpha_all, mx_v,
                   *sems):
    qrs = [qr_all.at[b] for b in range(_NBUF)]
    krs = [kr_all.at[b] for b in range(_NBUF)]
    sems_q = sems[:_NBUF]
    sems_k = sems[_NBUF:]
    c = lax.axis_index("c")
    s = lax.axis_index("s")
    wid = c * 16 + s
    base_g = wid * GT
    pltpu.sync_copy(src_hbm.at[pl.ds(base_g, GT)], src_v)
    pltpu.sync_copy(dst_hbm.at[pl.ds(base_g, GT)], dst_v)

    lane = lax.iota(jnp.int32, 16)

    def start(g, b):
        pltpu.make_async_copy(q_hbm.at[dst_v.at[g]], qrs[b], sems_q[b]).start()
        pltpu.make_async_copy(k_hbm.at[src_v.at[g]], krs[b], sems_k[b]).start()

    def wait(g, b):
        pltpu.make_async_copy(q_hbm.at[dst_v.at[g]], qrs[b], sems_q[b]).wait()
        pltpu.make_async_copy(k_hbm.at[src_v.at[g]], krs[b], sems_k[b]).wait()

    def compute(g, b, mx):
        qr = qrs[b]
        kr = krs[b]
        for sub in range(8):
            idx = lane + (sub * 16)
            acc = jnp.zeros((16,), jnp.float32)
            for ch in range(C):
                chv = jnp.full((16,), ch, jnp.int32)
                qc = plsc.load_gather(qr, [idx, chv])
                kc = plsc.load_gather(kr, [idx, chv])
                acc = acc + qc * kc
            acc = acc * INV_SQRT_C
            alpha_all[g, pl.ds(sub * 16, 16)] = acc
            mx = jnp.maximum(mx, acc)
        return mx

    for b in range(_NBUF - 1):
        start(b, b)

    def quad(it, mx):
        for j in range(_NBUF):
            g = _NBUF * it + j

            @pl.when(g + _NBUF - 1 < GT)
            def _():
                start(g + _NBUF - 1, (j + _NBUF - 1) % _NBUF)

            wait(g, j)
            mx = compute(g, j, mx)
        return mx

    mx = lax.fori_loop(0, GT // _NBUF, quad,
                       jnp.full((16,), -1e30, jnp.float32))
    pltpu.sync_copy(alpha_all, alpha_out.at[pl.ds(base_g, GT)])
    mx_v[...] = mx
    pltpu.sync_copy(mx_v, mx_out.at[wid])


@functools.lru_cache(maxsize=1)
def _sc_alpha():
    return pl.kernel(
        _sc_alpha_body,
        out_type=[
            jax.ShapeDtypeStruct((W * GT, GROUP), jnp.float32),
            jax.ShapeDtypeStruct((W, 16), jnp.float32),
        ],
        mesh=_sc_mesh(),
        scratch_types=[
            pltpu.VMEM((GT, GROUP), jnp.int32),
            pltpu.VMEM((GT, GROUP), jnp.int32),
            pltpu.VMEM((_NBUF, GROUP, CP), jnp.float32),
            pltpu.VMEM((_NBUF, GROUP, CP), jnp.float32),
            pltpu.VMEM((GT, GROUP), jnp.float32),
            pltpu.VMEM((16,), jnp.float32),
        ] + [pltpu.SemaphoreType.DMA] * (2 * _NBUF),
        compiler_params=pltpu.CompilerParams(use_tc_tiling_on_sc=False, needs_layout_passes=False),
    )


def _sc_message_body(v_hbm, alpha_hbm, mx_hbm, src_hbm, dst_hbm,
                     acc_out,
                     src_v, dst_v, alpha_vt, vr_all, msg_all, mxv, tmp,
                     acc_spmem, *sems):
    vrs = [vr_all.at[b] for b in range(_NBUF)]
    msgs = [msg_all.at[b] for b in range(_NBUF)]
    sems_v = sems[:_NBUF]
    sems_s = sems[_NBUF:]
    c = lax.axis_index("c")
    s = lax.axis_index("s")
    wid = c * 16 + s
    base_g = wid * GT

    pltpu.sync_copy(src_hbm.at[pl.ds(base_g, GT)], src_v)
    pltpu.sync_copy(dst_hbm.at[pl.ds(base_g, GT)], dst_v)
    pltpu.sync_copy(alpha_hbm.at[pl.ds(base_g, GT)], alpha_vt)
    pltpu.sync_copy(mx_hbm, mxv)

    m = jnp.full((16,), -1e30, jnp.float32)
    for i in range(W):
        m = jnp.maximum(m, mxv[i, :])
    gmax = jnp.max(m)

    lane = lax.iota(jnp.int32, 16)
    zero16 = jnp.zeros((16,), jnp.float32)

    def zrow(i, _):
        tmp[i, :] = zero16
        return 0
    lax.fori_loop(0, ROWS_PER_TILE, zrow, 0)

    def zmsg(i, _):
        for b in range(_NBUF):
            msgs[b][i, :] = zero16
        return 0
    lax.fori_loop(0, GROUP, zmsg, 0)

    pltpu.sync_copy(tmp, acc_spmem.at[pl.ds(s * ROWS_PER_TILE, ROWS_PER_TILE)])
    plsc.subcore_barrier()

    def start_v(g, b):
        pltpu.make_async_copy(v_hbm.at[src_v.at[g]], vrs[b], sems_v[b]).start()

    def wait_v(g, b):
        pltpu.make_async_copy(v_hbm.at[src_v.at[g]], vrs[b], sems_v[b]).wait()

    def wait_scatter(b):
        pltpu.make_async_copy(msgs[b], acc_spmem.at[dst_v.at[0]],
                              sems_s[b]).wait()

    def compute(g, b):
        vr = vrs[b]
        msg = msgs[b]
        for sub in range(8):
            idx = lane + (sub * 16)
            a = alpha_vt[g, pl.ds(sub * 16, 16)]
            ae = jnp.exp(a - gmax)
            gid = (base_g + g) * GROUP + sub * 16 + lane
            ae = jnp.where(gid < E, ae, 0.0)
            for ch in range(C):
                chv = jnp.full((16,), ch, jnp.int32)
                vc = plsc.load_gather(vr, [idx, chv])
                plsc.store_scatter(msg, [idx, chv], vc * ae)
            plsc.store_scatter(msg, [idx, jnp.full((16,), C, jnp.int32)], ae)
        pltpu.async_copy(msg, acc_spmem.at[dst_v.at[g]], sems_s[b], add=True)

    for b in range(_NBUF - 1):
        start_v(b, b)

    def quad(it, _):
        for j in range(_NBUF):
            g = _NBUF * it + j

            @pl.when(g + _NBUF - 1 < GT)
            def _():
                start_v(g + _NBUF - 1, (j + _NBUF - 1) % _NBUF)

            wait_v(g, j)

            @pl.when(it > 0)
            def _():
                wait_scatter(j)

            compute(g, j)
        return 0

    lax.fori_loop(0, GT // _NBUF, quad, 0)
    for b in range(_NBUF):
        wait_scatter(b)
    plsc.subcore_barrier()

    pltpu.sync_copy(acc_spmem.at[pl.ds(s * ROWS_PER_TILE, ROWS_PER_TILE)], tmp)
    pltpu.sync_copy(tmp, acc_out.at[c, pl.ds(s * ROWS_PER_TILE, ROWS_PER_TILE)])


@functools.lru_cache(maxsize=1)
def _sc_message():
    return pl.kernel(
        _sc_message_body,
        out_type=jax.ShapeDtypeStruct((2, N, CP), jnp.float32),
        mesh=_sc_mesh(),
        scratch_types=[
            pltpu.VMEM((GT, GROUP), jnp.int32),
            pltpu.VMEM((GT, GROUP), jnp.int32),
            pltpu.VMEM((GT, GROUP), jnp.float32),
            pltpu.VMEM((_NBUF, GROUP, CP), jnp.float32),
            pltpu.VMEM((_NBUF, GROUP, CP), jnp.float32),
            pltpu.VMEM((W, 16), jnp.float32),
            pltpu.VMEM((ROWS_PER_TILE, CP), jnp.float32),
            pltpu.VMEM_SHARED((N, CP), jnp.float32),
        ] + [pltpu.SemaphoreType.DMA] * (2 * _NBUF),
        compiler_params=pltpu.CompilerParams(use_tc_tiling_on_sc=False, needs_layout_passes=False),
    )


def _pack_weights(wq, bq, wk, bk, wv, bv, ws, bs, in_dim):
    wcat = jnp.zeros((in_dim, 4 * CP), jnp.float32)
    bcat = jnp.zeros((1, 4 * CP), jnp.float32)
    for slot, (w_, b_) in enumerate(((wq, bq), (wk, bk), (wv, bv), (ws, bs))):
        wcat = wcat.at[: w_.shape[0], slot * CP: slot * CP + C].set(w_)
        bcat = bcat.at[0, slot * CP: slot * CP + C].set(b_)
    return wcat, bcat


def kernel(x, edge_index, Wq1, bq1, Wk1, bk1, Wv1, bv1, Ws1, bs1,
           Wq2, bq2, Wk2, bk2, Wv2, bv2, Ws2, bs2, Wo, bo):
    src = jnp.pad(edge_index[0], (0, EP - E)).reshape(W * GT, GROUP)
    dst = jnp.pad(edge_index[1], (0, EP - E)).reshape(W * GT, GROUP)

    wcat1, bcat1 = _pack_weights(Wq1, bq1, Wk1, bk1, Wv1, bv1, Ws1, bs1, D)
    wcat2, bcat2 = _pack_weights(Wq2, bq2, Wk2, bk2, Wv2, bv2, Ws2, bs2, CP)
    wo_pad = jnp.zeros((CP, D), jnp.float32).at[:C, :].set(Wo)
    bo2 = bo.reshape(1, D)

    sc_alpha = _sc_alpha()
    sc_message = _sc_message()

    q1, k1, v1, s1 = _tc_project(x, wcat1, bcat1, D)
    alpha1, mx1 = sc_alpha(q1, k1, src, dst)
    acc1 = sc_message(v1, alpha1, mx1, src, dst)
    h1 = _tc_merge(acc1, s1)

    q2, k2, v2, s2 = _tc_project(h1, wcat2, bcat2, CP)
    alpha2, mx2 = sc_alpha(q2, k2, src, dst)
    acc2 = sc_message(v2, alpha2, mx2, src, dst)

    return _tc_final(acc2, s2, wo_pad, bo2)


# speedup vs baseline: 34.5662x; 1.0420x over previous
import functools

import jax
import jax.numpy as jnp
from jax import lax
from jax.experimental import pallas as pl
from jax.experimental.pallas import tpu as pltpu
from jax.experimental.pallas import tpu_sc as plsc

N = 10000
E = 320000
D = 128
C = 11
CP = 16
W = 32
GT = 80
GROUP = 128
EP = W * GT * GROUP
ROWS_PER_TILE = N // 16
INV_SQRT_C = 1.0 / (C ** 0.5)
_NBUF = 4


_BLK = 2000


def _project_body(x_ref, w_ref, b_ref, q_ref, k_ref, v_ref, s_ref):
    h = jnp.dot(x_ref[...], w_ref[...], preferred_element_type=jnp.float32)
    h = h + b_ref[...]
    q_ref[...] = h[:, 0:16]
    k_ref[...] = h[:, 16:32]
    v_ref[...] = h[:, 32:48]
    s_ref[...] = h[:, 48:64]


def _tc_project(x, wcat, bcat, in_dim):
    blk = _BLK
    grid = N // blk
    out = jax.ShapeDtypeStruct((N, CP), jnp.float32)
    return pl.pallas_call(
        _project_body,
        grid=(grid,),
        in_specs=[
            pl.BlockSpec((blk, in_dim), lambda i: (i, 0)),
            pl.BlockSpec((in_dim, 4 * CP), lambda i: (0, 0)),
            pl.BlockSpec((1, 4 * CP), lambda i: (0, 0)),
        ],
        out_specs=[pl.BlockSpec((blk, CP), lambda i: (i, 0))] * 4,
        out_shape=[out, out, out, out],
    )(x, wcat, bcat)


def _merge_scaled(acc_ref, mx_ref):
    g0 = jnp.max(mx_ref[0])
    g1 = jnp.max(mx_ref[1])
    g = jnp.maximum(g0, g1)
    return acc_ref[0] * jnp.exp(g0 - g) + acc_ref[1] * jnp.exp(g1 - g)


def _merge_body(acc_ref, mx_ref, skip_ref, h_ref):
    a = _merge_scaled(acc_ref, mx_ref)
    wsum = a[:, 11:12]
    h = a / (wsum + 1e-16) + skip_ref[...]
    h = jnp.maximum(h, 0.0)
    col = lax.broadcasted_iota(jnp.int32, h.shape, 1)
    h_ref[...] = jnp.where(col < C, h, 0.0)


def _tc_merge(acc, mx, skip):
    blk = _BLK
    return pl.pallas_call(
        _merge_body,
        grid=(N // blk,),
        in_specs=[
            pl.BlockSpec((2, blk, CP), lambda i: (0, i, 0)),
            pl.BlockSpec((2, 16), lambda i: (0, 0)),
            pl.BlockSpec((blk, CP), lambda i: (i, 0)),
        ],
        out_specs=pl.BlockSpec((blk, CP), lambda i: (i, 0)),
        out_shape=jax.ShapeDtypeStruct((N, CP), jnp.float32),
    )(acc, mx, skip)


def _final_body(acc_ref, mx_ref, skip_ref, wo_ref, bo_ref, y_ref):
    a = _merge_scaled(acc_ref, mx_ref)
    wsum = a[:, 11:12]
    h = a / (wsum + 1e-16) + skip_ref[...]
    h = jnp.maximum(h, 0.0)
    col = lax.broadcasted_iota(jnp.int32, h.shape, 1)
    h = jnp.where(col < C, h, 0.0)
    z = jnp.dot(h, wo_ref[...], preferred_element_type=jnp.float32)
    z = z + bo_ref[...]
    y_ref[...] = 1.0 / (1.0 + jnp.exp(-z))


def _tc_final(acc, mx, skip, wo_pad, bo):
    blk = _BLK
    return pl.pallas_call(
        _final_body,
        grid=(N // blk,),
        in_specs=[
            pl.BlockSpec((2, blk, CP), lambda i: (0, i, 0)),
            pl.BlockSpec((2, 16), lambda i: (0, 0)),
            pl.BlockSpec((blk, CP), lambda i: (i, 0)),
            pl.BlockSpec((CP, D), lambda i: (0, 0)),
            pl.BlockSpec((1, D), lambda i: (0, 0)),
        ],
        out_specs=pl.BlockSpec((blk, D), lambda i: (i, 0)),
        out_shape=jax.ShapeDtypeStruct((N, D), jnp.float32),
    )(acc, mx, skip, wo_pad, bo)


@functools.lru_cache(maxsize=1)
def _sc_mesh():
    return plsc.VectorSubcoreMesh(
        core_axis_name="c", subcore_axis_name="s", num_cores=2, num_subcores=16)


def _sc_layer_body(q_hbm, k_hbm, v_hbm, src_hbm, dst_hbm,
                   acc_out, mx_out,
                   src_v, dst_v, alpha_vt, qr_all, kr_all, vr_all, msg_all,
                   mxv, mx_v, tmp,
                   acc_spmem, mx_spmem, *sems):
    qrs = [qr_all.at[b] for b in range(_NBUF)]
    krs = [kr_all.at[b] for b in range(_NBUF)]
    vrs = [vr_all.at[b] for b in range(_NBUF)]
    msgs = [msg_all.at[b] for b in range(_NBUF)]
    sems_q = sems[0:_NBUF]
    sems_k = sems[_NBUF:2 * _NBUF]
    sems_v = sems[2 * _NBUF:3 * _NBUF]
    sems_s = sems[3 * _NBUF:4 * _NBUF]

    c = lax.axis_index("c")
    s = lax.axis_index("s")
    wid = c * 16 + s
    base_g = wid * GT
    lane = lax.iota(jnp.int32, 16)
    zero16 = jnp.zeros((16,), jnp.float32)

    pltpu.sync_copy(src_hbm.at[pl.ds(base_g, GT)], src_v)
    pltpu.sync_copy(dst_hbm.at[pl.ds(base_g, GT)], dst_v)

    def start_qk(g, b):
        pltpu.make_async_copy(q_hbm.at[dst_v.at[g]], qrs[b], sems_q[b]).start()
        pltpu.make_async_copy(k_hbm.at[src_v.at[g]], krs[b], sems_k[b]).start()

    def wait_qk(g, b):
        pltpu.make_async_copy(q_hbm.at[dst_v.at[g]], qrs[b], sems_q[b]).wait()
        pltpu.make_async_copy(k_hbm.at[src_v.at[g]], krs[b], sems_k[b]).wait()

    def compute_alpha(g, b, mx):
        qr = qrs[b]
        kr = krs[b]
        for sub in range(8):
            idx = lane + (sub * 16)
            acc = jnp.zeros((16,), jnp.float32)
            for ch in range(C):
                chv = jnp.full((16,), ch, jnp.int32)
                qc = plsc.load_gather(qr, [idx, chv])
                kc = plsc.load_gather(kr, [idx, chv])
                acc = acc + qc * kc
            acc = acc * INV_SQRT_C
            alpha_vt[g, pl.ds(sub * 16, 16)] = acc
            mx = jnp.maximum(mx, acc)
        return mx

    for b in range(_NBUF - 1):
        start_qk(b, b)

    def zrow(i, _):
        tmp[i, :] = zero16
        return 0
    lax.fori_loop(0, ROWS_PER_TILE, zrow, 0)

    def zmsg(i, _):
        for b in range(_NBUF):
            msgs[b][i, :] = zero16
        return 0
    lax.fori_loop(0, GROUP, zmsg, 0)

    pltpu.sync_copy(tmp, acc_spmem.at[pl.ds(s * ROWS_PER_TILE, ROWS_PER_TILE)])

    def quad1(it, mx):
        for j in range(_NBUF):
            g = _NBUF * it + j

            @pl.when(g + _NBUF - 1 < GT)
            def _():
                start_qk(g + _NBUF - 1, (j + _NBUF - 1) % _NBUF)

            wait_qk(g, j)
            mx = compute_alpha(g, j, mx)
        return mx

    mx = lax.fori_loop(0, GT // _NBUF, quad1,
                       jnp.full((16,), -1e30, jnp.float32))

    mx_v[...] = mx
    pltpu.sync_copy(mx_v, mx_spmem.at[s])

    def start_v(g, b):
        pltpu.make_async_copy(v_hbm.at[src_v.at[g]], vrs[b], sems_v[b]).start()

    for b in range(_NBUF - 1):
        start_v(b, b)

    plsc.subcore_barrier()

    pltpu.sync_copy(mx_spmem, mxv)
    m = jnp.full((16,), -1e30, jnp.float32)
    for i in range(16):
        m = jnp.maximum(m, mxv[i, :])
    gmax = jnp.max(m)

    @pl.when(s == 0)
    def _():
        mx_v[...] = m
        pltpu.sync_copy(mx_v, mx_out.at[c])

    def wait_v(g, b):
        pltpu.make_async_copy(v_hbm.at[src_v.at[g]], vrs[b], sems_v[b]).wait()

    def wait_scatter(b):
        pltpu.make_async_copy(msgs[b], acc_spmem.at[dst_v.at[0]],
                              sems_s[b]).wait()

    def compute_msg(g, b):
        vr = vrs[b]
        msg = msgs[b]
        for sub in range(8):
            idx = lane + (sub * 16)
            a = alpha_vt[g, pl.ds(sub * 16, 16)]
            ae = jnp.exp(a - gmax)
            gid = (base_g + g) * GROUP + sub * 16 + lane
            ae = jnp.where(gid < E, ae, 0.0)
            for ch in range(C):
                chv = jnp.full((16,), ch, jnp.int32)
                vc = plsc.load_gather(vr, [idx, chv])
                plsc.store_scatter(msg, [idx, chv], vc * ae)
            plsc.store_scatter(msg, [idx, jnp.full((16,), C, jnp.int32)], ae)
        pltpu.async_copy(msg, acc_spmem.at[dst_v.at[g]], sems_s[b], add=True)

    def quad2(it, _):
        for j in range(_NBUF):
            g = _NBUF * it + j

            @pl.when(g + _NBUF - 1 < GT)
            def _():
                start_v(g + _NBUF - 1, (j + _NBUF - 1) % _NBUF)

            wait_v(g, j)

            @pl.when(it > 0)
            def _():
                wait_scatter(j)

            compute_msg(g, j)
        return 0

    lax.fori_loop(0, GT // _NBUF, quad2, 0)
    for b in range(_NBUF):
        wait_scatter(b)
    plsc.subcore_barrier()

    pltpu.sync_copy(acc_spmem.at[pl.ds(s * ROWS_PER_TILE, ROWS_PER_TILE)], tmp)
    pltpu.sync_copy(tmp, acc_out.at[c, pl.ds(s * ROWS_PER_TILE, ROWS_PER_TILE)])


@functools.lru_cache(maxsize=1)
def _sc_layer():
    return pl.kernel(
        _sc_layer_body,
        out_type=[
            jax.ShapeDtypeStruct((2, N, CP), jnp.float32),
            jax.ShapeDtypeStruct((2, 16), jnp.float32),
        ],
        mesh=_sc_mesh(),
        scratch_types=[
            pltpu.VMEM((GT, GROUP), jnp.int32),
            pltpu.VMEM((GT, GROUP), jnp.int32),
            pltpu.VMEM((GT, GROUP), jnp.float32),
            pltpu.VMEM((_NBUF, GROUP, CP), jnp.float32),
            pltpu.VMEM((_NBUF, GROUP, CP), jnp.float32),
            pltpu.VMEM((_NBUF, GROUP, CP), jnp.float32),
            pltpu.VMEM((_NBUF, GROUP, CP), jnp.float32),
            pltpu.VMEM((16, 16), jnp.float32),
            pltpu.VMEM((16,), jnp.float32),
            pltpu.VMEM((ROWS_PER_TILE, CP), jnp.float32),
            pltpu.VMEM_SHARED((N, CP), jnp.float32),
            pltpu.VMEM_SHARED((16, 16), jnp.float32),
        ] + [pltpu.SemaphoreType.DMA] * (4 * _NBUF),
        compiler_params=pltpu.CompilerParams(
            use_tc_tiling_on_sc=False, needs_layout_passes=False),
    )


def _pack_weights(wq, bq, wk, bk, wv, bv, ws, bs, in_dim):
    wcat = jnp.zeros((in_dim, 4 * CP), jnp.float32)
    bcat = jnp.zeros((1, 4 * CP), jnp.float32)
    for slot, (w_, b_) in enumerate(((wq, bq), (wk, bk), (wv, bv), (ws, bs))):
        wcat = wcat.at[: w_.shape[0], slot * CP: slot * CP + C].set(w_)
        bcat = bcat.at[0, slot * CP: slot * CP + C].set(b_)
    return wcat, bcat


def kernel(x, edge_index, Wq1, bq1, Wk1, bk1, Wv1, bv1, Ws1, bs1,
           Wq2, bq2, Wk2, bk2, Wv2, bv2, Ws2, bs2, Wo, bo):
    src = jnp.pad(edge_index[0], (0, EP - E)).reshape(W * GT, GROUP)
    dst = jnp.pad(edge_index[1], (0, EP - E)).reshape(W * GT, GROUP)

    wcat1, bcat1 = _pack_weights(Wq1, bq1, Wk1, bk1, Wv1, bv1, Ws1, bs1, D)
    wcat2, bcat2 = _pack_weights(Wq2, bq2, Wk2, bk2, Wv2, bv2, Ws2, bs2, CP)
    wo_pad = jnp.zeros((CP, D), jnp.float32).at[:C, :].set(Wo)
    bo2 = bo.reshape(1, D)

    sc_layer = _sc_layer()

    q1, k1, v1, s1 = _tc_project(x, wcat1, bcat1, D)
    acc1, mx1 = sc_layer(q1, k1, v1, src, dst)
    h1 = _tc_merge(acc1, mx1, s1)

    q2, k2, v2, s2 = _tc_project(h1, wcat2, bcat2, CP)
    acc2, mx2 = sc_layer(q2, k2, v2, src, dst)

    return _tc_final(acc2, mx2, s2, wo_pad, bo2)


# speedup vs baseline: 38.6022x; 1.1168x over previous
import functools

import jax
import jax.numpy as jnp
from jax import lax
from jax.experimental import pallas as pl
from jax.experimental.pallas import tpu as pltpu
from jax.experimental.pallas import tpu_sc as plsc

N = 10000
E = 320000
D = 128
C = 11
CP = 16
W = 32
GT = 80
GROUP = 128
EP = W * GT * GROUP
ROWS_PER_TILE = N // 16
INV_SQRT_C = 1.0 / (C ** 0.5)
_NBUF = 4


_BLK = 2000


def _project_body(x_ref, w_ref, b_ref, q_ref, k_ref, v_ref, s_ref):
    h = jnp.dot(x_ref[...], w_ref[...], preferred_element_type=jnp.float32)
    h = h + b_ref[...]
    q_ref[...] = h[:, 0:16]
    k_ref[...] = h[:, 16:32]
    v_ref[...] = h[:, 32:48]
    s_ref[...] = h[:, 48:64]


def _tc_project(x, wcat, bcat, in_dim):
    blk = _BLK
    grid = N // blk
    out = jax.ShapeDtypeStruct((N, CP), jnp.float32)
    return pl.pallas_call(
        _project_body,
        grid=(grid,),
        in_specs=[
            pl.BlockSpec((blk, in_dim), lambda i: (i, 0)),
            pl.BlockSpec((in_dim, 4 * CP), lambda i: (0, 0)),
            pl.BlockSpec((1, 4 * CP), lambda i: (0, 0)),
        ],
        out_specs=[pl.BlockSpec((blk, CP), lambda i: (i, 0))] * 4,
        out_shape=[out, out, out, out],
    )(x, wcat, bcat)


def _merge_scaled(acc_ref, mx_ref):
    g0 = jnp.max(mx_ref[0])
    g1 = jnp.max(mx_ref[1])
    g = jnp.maximum(g0, g1)
    return acc_ref[0] * jnp.exp(g0 - g) + acc_ref[1] * jnp.exp(g1 - g)


def _merge_body(acc_ref, mx_ref, skip_ref, h_ref):
    a = _merge_scaled(acc_ref, mx_ref)
    wsum = a[:, 11:12]
    h = a / (wsum + 1e-16) + skip_ref[...]
    h = jnp.maximum(h, 0.0)
    col = lax.broadcasted_iota(jnp.int32, h.shape, 1)
    h_ref[...] = jnp.where(col < C, h, 0.0)


def _tc_merge(acc, mx, skip):
    blk = _BLK
    return pl.pallas_call(
        _merge_body,
        grid=(N // blk,),
        in_specs=[
            pl.BlockSpec((2, blk, CP), lambda i: (0, i, 0)),
            pl.BlockSpec((2, 16), lambda i: (0, 0)),
            pl.BlockSpec((blk, CP), lambda i: (i, 0)),
        ],
        out_specs=pl.BlockSpec((blk, CP), lambda i: (i, 0)),
        out_shape=jax.ShapeDtypeStruct((N, CP), jnp.float32),
    )(acc, mx, skip)


def _final_body(acc_ref, mx_ref, skip_ref, wo_ref, bo_ref, y_ref):
    a = _merge_scaled(acc_ref, mx_ref)
    wsum = a[:, 11:12]
    h = a / (wsum + 1e-16) + skip_ref[...]
    h = jnp.maximum(h, 0.0)
    col = lax.broadcasted_iota(jnp.int32, h.shape, 1)
    h = jnp.where(col < C, h, 0.0)
    z = jnp.dot(h, wo_ref[...], preferred_element_type=jnp.float32)
    z = z + bo_ref[...]
    y_ref[...] = 1.0 / (1.0 + jnp.exp(-z))


def _tc_final(acc, mx, skip, wo_pad, bo):
    blk = _BLK
    return pl.pallas_call(
        _final_body,
        grid=(N // blk,),
        in_specs=[
            pl.BlockSpec((2, blk, CP), lambda i: (0, i, 0)),
            pl.BlockSpec((2, 16), lambda i: (0, 0)),
            pl.BlockSpec((blk, CP), lambda i: (i, 0)),
            pl.BlockSpec((CP, D), lambda i: (0, 0)),
            pl.BlockSpec((1, D), lambda i: (0, 0)),
        ],
        out_specs=pl.BlockSpec((blk, D), lambda i: (i, 0)),
        out_shape=jax.ShapeDtypeStruct((N, D), jnp.float32),
    )(acc, mx, skip, wo_pad, bo)


@functools.lru_cache(maxsize=1)
def _sc_mesh():
    return plsc.VectorSubcoreMesh(
        core_axis_name="c", subcore_axis_name="s", num_cores=2, num_subcores=16)


def _sc_layer_body(q_hbm, k_hbm, v_hbm, ei_hbm,
                   acc_out, mx_out,
                   src_v, dst_v, alpha_vt, qr_all, kr_all, vr_all, msg_all,
                   mxv, mx_v, tmp,
                   acc_spmem, mx_spmem, *sems):
    qrs = [qr_all.at[b] for b in range(_NBUF)]
    krs = [kr_all.at[b] for b in range(_NBUF)]
    vrs = [vr_all.at[b] for b in range(_NBUF)]
    msgs = [msg_all.at[b] for b in range(_NBUF)]
    sems_q = sems[0:_NBUF]
    sems_k = sems[_NBUF:2 * _NBUF]
    sems_v = sems[2 * _NBUF:3 * _NBUF]
    sems_s = sems[3 * _NBUF:4 * _NBUF]

    c = lax.axis_index("c")
    s = lax.axis_index("s")
    wid = c * 16 + s
    base_g = wid * GT
    lane = lax.iota(jnp.int32, 16)
    zero16 = jnp.zeros((16,), jnp.float32)

    pltpu.sync_copy(ei_hbm.at[0, pl.ds(base_g, GT)], src_v)
    pltpu.sync_copy(ei_hbm.at[1, pl.ds(base_g, GT)], dst_v)

    def start_qk(g, b):
        pltpu.make_async_copy(q_hbm.at[dst_v.at[g]], qrs[b], sems_q[b]).start()
        pltpu.make_async_copy(k_hbm.at[src_v.at[g]], krs[b], sems_k[b]).start()

    def wait_qk(g, b):
        pltpu.make_async_copy(q_hbm.at[dst_v.at[g]], qrs[b], sems_q[b]).wait()
        pltpu.make_async_copy(k_hbm.at[src_v.at[g]], krs[b], sems_k[b]).wait()

    def compute_alpha(g, b, mx):
        qr = qrs[b]
        kr = krs[b]
        for sub in range(8):
            idx = lane + (sub * 16)
            acc = jnp.zeros((16,), jnp.float32)
            for ch in range(C):
                chv = jnp.full((16,), ch, jnp.int32)
                qc = plsc.load_gather(qr, [idx, chv])
                kc = plsc.load_gather(kr, [idx, chv])
                acc = acc + qc * kc
            acc = acc * INV_SQRT_C
            alpha_vt[g, pl.ds(sub * 16, 16)] = acc
            mx = jnp.maximum(mx, acc)
        return mx

    for b in range(_NBUF - 1):
        start_qk(b, b)

    def zrow(i, _):
        tmp[i, :] = zero16
        return 0
    lax.fori_loop(0, ROWS_PER_TILE, zrow, 0)

    def zmsg(i, _):
        for b in range(_NBUF):
            msgs[b][i, :] = zero16
        return 0
    lax.fori_loop(0, GROUP, zmsg, 0)

    pltpu.sync_copy(tmp, acc_spmem.at[pl.ds(s * ROWS_PER_TILE, ROWS_PER_TILE)])

    def quad1(it, mx):
        for j in range(_NBUF):
            g = _NBUF * it + j

            @pl.when(g + _NBUF - 1 < GT)
            def _():
                start_qk(g + _NBUF - 1, (j + _NBUF - 1) % _NBUF)

            wait_qk(g, j)
            mx = compute_alpha(g, j, mx)
        return mx

    mx = lax.fori_loop(0, GT // _NBUF, quad1,
                       jnp.full((16,), -1e30, jnp.float32))

    mx_v[...] = mx
    pltpu.sync_copy(mx_v, mx_spmem.at[s])

    def start_v(g, b):
        pltpu.make_async_copy(v_hbm.at[src_v.at[g]], vrs[b], sems_v[b]).start()

    for b in range(_NBUF - 1):
        start_v(b, b)

    plsc.subcore_barrier()

    pltpu.sync_copy(mx_spmem, mxv)
    m = jnp.full((16,), -1e30, jnp.float32)
    for i in range(16):
        m = jnp.maximum(m, mxv[i, :])
    gmax = jnp.max(m)

    @pl.when(s == 0)
    def _():
        mx_v[...] = m
        pltpu.sync_copy(mx_v, mx_out.at[c])

    def wait_v(g, b):
        pltpu.make_async_copy(v_hbm.at[src_v.at[g]], vrs[b], sems_v[b]).wait()

    def wait_scatter(b):
        pltpu.make_async_copy(msgs[b], acc_spmem.at[dst_v.at[0]],
                              sems_s[b]).wait()

    def compute_msg(g, b):
        vr = vrs[b]
        msg = msgs[b]
        for sub in range(8):
            idx = lane + (sub * 16)
            a = alpha_vt[g, pl.ds(sub * 16, 16)]
            ae = jnp.exp(a - gmax)
            gid = (base_g + g) * GROUP + sub * 16 + lane
            ae = jnp.where(gid < E, ae, 0.0)
            for ch in range(C):
                chv = jnp.full((16,), ch, jnp.int32)
                vc = plsc.load_gather(vr, [idx, chv])
                plsc.store_scatter(msg, [idx, chv], vc * ae)
            plsc.store_scatter(msg, [idx, jnp.full((16,), C, jnp.int32)], ae)
        pltpu.async_copy(msg, acc_spmem.at[dst_v.at[g]], sems_s[b], add=True)

    def quad2(it, _):
        for j in range(_NBUF):
            g = _NBUF * it + j

            @pl.when(g + _NBUF - 1 < GT)
            def _():
                start_v(g + _NBUF - 1, (j + _NBUF - 1) % _NBUF)

            wait_v(g, j)

            @pl.when(it > 0)
            def _():
                wait_scatter(j)

            compute_msg(g, j)
        return 0

    lax.fori_loop(0, GT // _NBUF, quad2, 0)
    for b in range(_NBUF):
        wait_scatter(b)
    plsc.subcore_barrier()

    pltpu.sync_copy(acc_spmem.at[pl.ds(s * ROWS_PER_TILE, ROWS_PER_TILE)], tmp)
    pltpu.sync_copy(tmp, acc_out.at[c, pl.ds(s * ROWS_PER_TILE, ROWS_PER_TILE)])


@functools.lru_cache(maxsize=1)
def _sc_layer():
    return pl.kernel(
        _sc_layer_body,
        out_type=[
            jax.ShapeDtypeStruct((2, N, CP), jnp.float32),
            jax.ShapeDtypeStruct((2, 16), jnp.float32),
        ],
        mesh=_sc_mesh(),
        scratch_types=[
            pltpu.VMEM((GT, GROUP), jnp.int32),
            pltpu.VMEM((GT, GROUP), jnp.int32),
            pltpu.VMEM((GT, GROUP), jnp.float32),
            pltpu.VMEM((_NBUF, GROUP, CP), jnp.float32),
            pltpu.VMEM((_NBUF, GROUP, CP), jnp.float32),
            pltpu.VMEM((_NBUF, GROUP, CP), jnp.float32),
            pltpu.VMEM((_NBUF, GROUP, CP), jnp.float32),
            pltpu.VMEM((16, 16), jnp.float32),
            pltpu.VMEM((16,), jnp.float32),
            pltpu.VMEM((ROWS_PER_TILE, CP), jnp.float32),
            pltpu.VMEM_SHARED((N, CP), jnp.float32),
            pltpu.VMEM_SHARED((16, 16), jnp.float32),
        ] + [pltpu.SemaphoreType.DMA] * (4 * _NBUF),
        compiler_params=pltpu.CompilerParams(
            use_tc_tiling_on_sc=False, needs_layout_passes=False),
    )


def _pack_weights(wq, bq, wk, bk, wv, bv, ws, bs, in_dim):
    wcat = jnp.pad(jnp.stack([wq, wk, wv, ws], axis=1),
                   ((0, in_dim - wq.shape[0]), (0, 0),
                    (0, CP - C))).reshape(in_dim, 4 * CP)
    bcat = jnp.pad(jnp.stack([bq, bk, bv, bs], axis=0),
                   ((0, 0), (0, CP - C))).reshape(1, 4 * CP)
    return wcat, bcat


def kernel(x, edge_index, Wq1, bq1, Wk1, bk1, Wv1, bv1, Ws1, bs1,
           Wq2, bq2, Wk2, bk2, Wv2, bv2, Ws2, bs2, Wo, bo):
    ei = jnp.pad(edge_index, ((0, 0), (0, EP - E))).reshape(2, W * GT, GROUP)

    wcat1, bcat1 = _pack_weights(Wq1, bq1, Wk1, bk1, Wv1, bv1, Ws1, bs1, D)
    wcat2, bcat2 = _pack_weights(Wq2, bq2, Wk2, bk2, Wv2, bv2, Ws2, bs2, CP)
    wo_pad = jnp.pad(Wo, ((0, CP - C), (0, 0)))
    bo2 = bo.reshape(1, D)

    sc_layer = _sc_layer()

    q1, k1, v1, s1 = _tc_project(x, wcat1, bcat1, D)
    acc1, mx1 = sc_layer(q1, k1, v1, ei)
    h1 = _tc_merge(acc1, mx1, s1)

    q2, k2, v2, s2 = _tc_project(h1, wcat2, bcat2, CP)
    acc2, mx2 = sc_layer(q2, k2, v2, ei)

    return _tc_final(acc2, mx2, s2, wo_pad, bo2)


# speedup vs baseline: 39.2096x; 1.0157x over previous
import functools

import jax
import jax.numpy as jnp
from jax import lax
from jax.experimental import pallas as pl
from jax.experimental.pallas import tpu as pltpu
from jax.experimental.pallas import tpu_sc as plsc

N = 10000
E = 320000
D = 128
C = 11
CP = 16
W = 32
GT = 80
GROUP = 128
EP = W * GT * GROUP
ROWS_PER_TILE = N // 16
INV_SQRT_C = 1.0 / (C ** 0.5)
_NBUF = 4


_BLK = 2000


def _project_body(x_ref, w_ref, b_ref, q_ref, k_ref, v_ref, s_ref):
    h = jnp.dot(x_ref[...], w_ref[...], preferred_element_type=jnp.float32)
    h = h + b_ref[...]
    q_ref[...] = h[:, 0:16]
    k_ref[...] = h[:, 16:32]
    v_ref[...] = h[:, 32:48]
    s_ref[...] = h[:, 48:64]


def _tc_project(x, wcat, bcat, in_dim):
    blk = _BLK
    grid = N // blk
    out = jax.ShapeDtypeStruct((N, CP), jnp.float32)
    return pl.pallas_call(
        _project_body,
        grid=(grid,),
        in_specs=[
            pl.BlockSpec((blk, in_dim), lambda i: (i, 0)),
            pl.BlockSpec((in_dim, 4 * CP), lambda i: (0, 0)),
            pl.BlockSpec((1, 4 * CP), lambda i: (0, 0)),
        ],
        out_specs=[pl.BlockSpec((blk, CP), lambda i: (i, 0))] * 4,
        out_shape=[out, out, out, out],
    )(x, wcat, bcat)


def _merge_scaled(acc_ref, mx_ref):
    g0 = jnp.max(mx_ref[0])
    g1 = jnp.max(mx_ref[1])
    g = jnp.maximum(g0, g1)
    return acc_ref[0] * jnp.exp(g0 - g) + acc_ref[1] * jnp.exp(g1 - g)


def _mid_body(acc_ref, mx_ref, skip_ref, w_ref, b_ref,
              q_ref, k_ref, v_ref, s_ref):
    a = _merge_scaled(acc_ref, mx_ref)
    wsum = a[:, 11:12]
    h = a / (wsum + 1e-16) + skip_ref[...]
    h = jnp.maximum(h, 0.0)
    col = lax.broadcasted_iota(jnp.int32, h.shape, 1)
    h = jnp.where(col < C, h, 0.0)
    z = jnp.dot(h, w_ref[...], preferred_element_type=jnp.float32)
    z = z + b_ref[...]
    q_ref[...] = z[:, 0:16]
    k_ref[...] = z[:, 16:32]
    v_ref[...] = z[:, 32:48]
    s_ref[...] = z[:, 48:64]


def _tc_mid(acc, mx, skip, wcat, bcat):
    blk = _BLK
    out = jax.ShapeDtypeStruct((N, CP), jnp.float32)
    return pl.pallas_call(
        _mid_body,
        grid=(N // blk,),
        in_specs=[
            pl.BlockSpec((2, blk, CP), lambda i: (0, i, 0)),
            pl.BlockSpec((2, 16), lambda i: (0, 0)),
            pl.BlockSpec((blk, CP), lambda i: (i, 0)),
            pl.BlockSpec((CP, 4 * CP), lambda i: (0, 0)),
            pl.BlockSpec((1, 4 * CP), lambda i: (0, 0)),
        ],
        out_specs=[pl.BlockSpec((blk, CP), lambda i: (i, 0))] * 4,
        out_shape=[out, out, out, out],
    )(acc, mx, skip, wcat, bcat)


def _final_body(acc_ref, mx_ref, skip_ref, wo_ref, bo_ref, y_ref):
    a = _merge_scaled(acc_ref, mx_ref)
    wsum = a[:, 11:12]
    h = a / (wsum + 1e-16) + skip_ref[...]
    h = jnp.maximum(h, 0.0)
    col = lax.broadcasted_iota(jnp.int32, h.shape, 1)
    h = jnp.where(col < C, h, 0.0)
    z = jnp.dot(h, wo_ref[...], preferred_element_type=jnp.float32)
    z = z + bo_ref[...]
    y_ref[...] = 1.0 / (1.0 + jnp.exp(-z))


def _tc_final(acc, mx, skip, wo_pad, bo):
    blk = _BLK
    return pl.pallas_call(
        _final_body,
        grid=(N // blk,),
        in_specs=[
            pl.BlockSpec((2, blk, CP), lambda i: (0, i, 0)),
            pl.BlockSpec((2, 16), lambda i: (0, 0)),
            pl.BlockSpec((blk, CP), lambda i: (i, 0)),
            pl.BlockSpec((CP, D), lambda i: (0, 0)),
            pl.BlockSpec((1, D), lambda i: (0, 0)),
        ],
        out_specs=pl.BlockSpec((blk, D), lambda i: (i, 0)),
        out_shape=jax.ShapeDtypeStruct((N, D), jnp.float32),
    )(acc, mx, skip, wo_pad, bo)


@functools.lru_cache(maxsize=1)
def _sc_mesh():
    return plsc.VectorSubcoreMesh(
        core_axis_name="c", subcore_axis_name="s", num_cores=2, num_subcores=16)


def _sc_layer_body(q_hbm, k_hbm, v_hbm, ei_hbm,
                   acc_out, mx_out,
                   src_v, dst_v, alpha_vt, qr_all, kr_all, vr_all, msg_all,
                   mxv, mx_v, tmp,
                   acc_spmem, mx_spmem, *sems):
    qrs = [qr_all.at[b] for b in range(_NBUF)]
    krs = [kr_all.at[b] for b in range(_NBUF)]
    vrs = [vr_all.at[b] for b in range(_NBUF)]
    msgs = [msg_all.at[b] for b in range(_NBUF)]
    sems_q = sems[0:_NBUF]
    sems_k = sems[_NBUF:2 * _NBUF]
    sems_v = sems[2 * _NBUF:3 * _NBUF]
    sems_s = sems[3 * _NBUF:4 * _NBUF]

    c = lax.axis_index("c")
    s = lax.axis_index("s")
    wid = c * 16 + s
    base_g = wid * GT
    lane = lax.iota(jnp.int32, 16)
    zero16 = jnp.zeros((16,), jnp.float32)

    pltpu.sync_copy(ei_hbm.at[0, pl.ds(base_g, GT)], src_v)
    pltpu.sync_copy(ei_hbm.at[1, pl.ds(base_g, GT)], dst_v)

    def start_qk(g, b):
        pltpu.make_async_copy(q_hbm.at[dst_v.at[g]], qrs[b], sems_q[b]).start()
        pltpu.make_async_copy(k_hbm.at[src_v.at[g]], krs[b], sems_k[b]).start()

    def wait_qk(g, b):
        pltpu.make_async_copy(q_hbm.at[dst_v.at[g]], qrs[b], sems_q[b]).wait()
        pltpu.make_async_copy(k_hbm.at[src_v.at[g]], krs[b], sems_k[b]).wait()

    def compute_alpha(g, b, mx):
        qr = qrs[b]
        kr = krs[b]
        for sub in range(8):
            idx = lane + (sub * 16)
            acc = jnp.zeros((16,), jnp.float32)
            for ch in range(C):
                chv = jnp.full((16,), ch, jnp.int32)
                qc = plsc.load_gather(qr, [idx, chv])
                kc = plsc.load_gather(kr, [idx, chv])
                acc = acc + qc * kc
            acc = acc * INV_SQRT_C
            alpha_vt[g, pl.ds(sub * 16, 16)] = acc
            mx = jnp.maximum(mx, acc)
        return mx

    for b in range(_NBUF - 1):
        start_qk(b, b)

    def zrow(i, _):
        tmp[i, :] = zero16
        return 0
    lax.fori_loop(0, ROWS_PER_TILE, zrow, 0)

    def zmsg(i, _):
        for b in range(_NBUF):
            msgs[b][i, :] = zero16
        return 0
    lax.fori_loop(0, GROUP, zmsg, 0)

    pltpu.sync_copy(tmp, acc_spmem.at[pl.ds(s * ROWS_PER_TILE, ROWS_PER_TILE)])

    def quad1(it, mx):
        for j in range(_NBUF):
            g = _NBUF * it + j

            @pl.when(g + _NBUF - 1 < GT)
            def _():
                start_qk(g + _NBUF - 1, (j + _NBUF - 1) % _NBUF)

            wait_qk(g, j)
            mx = compute_alpha(g, j, mx)
        return mx

    mx = lax.fori_loop(0, GT // _NBUF, quad1,
                       jnp.full((16,), -1e30, jnp.float32))

    mx_v[...] = mx
    pltpu.sync_copy(mx_v, mx_spmem.at[s])

    def start_v(g, b):
        pltpu.make_async_copy(v_hbm.at[src_v.at[g]], vrs[b], sems_v[b]).start()

    for b in range(_NBUF - 1):
        start_v(b, b)

    plsc.subcore_barrier()

    pltpu.sync_copy(mx_spmem, mxv)
    m = jnp.full((16,), -1e30, jnp.float32)
    for i in range(16):
        m = jnp.maximum(m, mxv[i, :])
    gmax = jnp.max(m)

    @pl.when(s == 0)
    def _():
        mx_v[...] = m
        pltpu.sync_copy(mx_v, mx_out.at[c])

    def wait_v(g, b):
        pltpu.make_async_copy(v_hbm.at[src_v.at[g]], vrs[b], sems_v[b]).wait()

    def wait_scatter(b):
        pltpu.make_async_copy(msgs[b], acc_spmem.at[dst_v.at[0]],
                              sems_s[b]).wait()

    def compute_msg(g, b):
        vr = vrs[b]
        msg = msgs[b]
        for sub in range(8):
            idx = lane + (sub * 16)
            a = alpha_vt[g, pl.ds(sub * 16, 16)]
            ae = jnp.exp(a - gmax)
            gid = (base_g + g) * GROUP + sub * 16 + lane
            ae = jnp.where(gid < E, ae, 0.0)
            for ch in range(C):
                chv = jnp.full((16,), ch, jnp.int32)
                vc = plsc.load_gather(vr, [idx, chv])
                plsc.store_scatter(msg, [idx, chv], vc * ae)
            plsc.store_scatter(msg, [idx, jnp.full((16,), C, jnp.int32)], ae)
        pltpu.async_copy(msg, acc_spmem.at[dst_v.at[g]], sems_s[b], add=True)

    def quad2(it, _):
        for j in range(_NBUF):
            g = _NBUF * it + j

            @pl.when(g + _NBUF - 1 < GT)
            def _():
                start_v(g + _NBUF - 1, (j + _NBUF - 1) % _NBUF)

            wait_v(g, j)

            @pl.when(it > 0)
            def _():
                wait_scatter(j)

            compute_msg(g, j)
        return 0

    lax.fori_loop(0, GT // _NBUF, quad2, 0)
    for b in range(_NBUF):
        wait_scatter(b)
    plsc.subcore_barrier()

    pltpu.sync_copy(acc_spmem.at[pl.ds(s * ROWS_PER_TILE, ROWS_PER_TILE)], tmp)
    pltpu.sync_copy(tmp, acc_out.at[c, pl.ds(s * ROWS_PER_TILE, ROWS_PER_TILE)])


@functools.lru_cache(maxsize=1)
def _sc_layer():
    return pl.kernel(
        _sc_layer_body,
        out_type=[
            jax.ShapeDtypeStruct((2, N, CP), jnp.float32),
            jax.ShapeDtypeStruct((2, 16), jnp.float32),
        ],
        mesh=_sc_mesh(),
        scratch_types=[
            pltpu.VMEM((GT, GROUP), jnp.int32),
            pltpu.VMEM((GT, GROUP), jnp.int32),
            pltpu.VMEM((GT, GROUP), jnp.float32),
            pltpu.VMEM((_NBUF, GROUP, CP), jnp.float32),
            pltpu.VMEM((_NBUF, GROUP, CP), jnp.float32),
            pltpu.VMEM((_NBUF, GROUP, CP), jnp.float32),
            pltpu.VMEM((_NBUF, GROUP, CP), jnp.float32),
            pltpu.VMEM((16, 16), jnp.float32),
            pltpu.VMEM((16,), jnp.float32),
            pltpu.VMEM((ROWS_PER_TILE, CP), jnp.float32),
            pltpu.VMEM_SHARED((N, CP), jnp.float32),
            pltpu.VMEM_SHARED((16, 16), jnp.float32),
        ] + [pltpu.SemaphoreType.DMA] * (4 * _NBUF),
        compiler_params=pltpu.CompilerParams(
            use_tc_tiling_on_sc=False, needs_layout_passes=False),
    )


def _pack_weights(wq, bq, wk, bk, wv, bv, ws, bs, in_dim):
    wcat = jnp.pad(jnp.stack([wq, wk, wv, ws], axis=1),
                   ((0, in_dim - wq.shape[0]), (0, 0),
                    (0, CP - C))).reshape(in_dim, 4 * CP)
    bcat = jnp.pad(jnp.stack([bq, bk, bv, bs], axis=0),
                   ((0, 0), (0, CP - C))).reshape(1, 4 * CP)
    return wcat, bcat


def kernel(x, edge_index, Wq1, bq1, Wk1, bk1, Wv1, bv1, Ws1, bs1,
           Wq2, bq2, Wk2, bk2, Wv2, bv2, Ws2, bs2, Wo, bo):
    ei = jnp.pad(edge_index, ((0, 0), (0, EP - E))).reshape(2, W * GT, GROUP)

    wcat1, bcat1 = _pack_weights(Wq1, bq1, Wk1, bk1, Wv1, bv1, Ws1, bs1, D)
    wcat2, bcat2 = _pack_weights(Wq2, bq2, Wk2, bk2, Wv2, bv2, Ws2, bs2, CP)
    wo_pad = jnp.pad(Wo, ((0, CP - C), (0, 0)))
    bo2 = bo.reshape(1, D)

    sc_layer = _sc_layer()

    q1, k1, v1, s1 = _tc_project(x, wcat1, bcat1, D)
    acc1, mx1 = sc_layer(q1, k1, v1, ei)

    q2, k2, v2, s2 = _tc_mid(acc1, mx1, s1, wcat2, bcat2)
    acc2, mx2 = sc_layer(q2, k2, v2, ei)

    return _tc_final(acc2, mx2, s2, wo_pad, bo2)


# speedup vs baseline: 40.6570x; 1.0369x over previous
import functools

import jax
import jax.numpy as jnp
from jax import lax
from jax.experimental import pallas as pl
from jax.experimental.pallas import tpu as pltpu
from jax.experimental.pallas import tpu_sc as plsc

N = 10000
E = 320000
D = 128
C = 11
CP = 16
W = 32
GT = 80
GROUP = 128
EP = W * GT * GROUP
ROWS_PER_TILE = N // 16
INV_SQRT_C = 1.0 / (C ** 0.5)
_NBUF = 4


_BLK = 2000


_R = 8
_NB = N // _R
_BB = _NB

import numpy as _np
_SEL_NP = _np.zeros((_R * CP, _R * CP), _np.float32)
for _j in range(_R * CP):
    _SEL_NP[(_j // CP) * CP + C, _j] = 1.0
_COLMASK_NP = (_np.arange(_R * CP) % CP < C).astype(_np.float32)


def _project_body(x_ref, w_ref, b_ref, q_ref, k_ref, v_ref, s_ref):
    z = jnp.dot(x_ref[...], w_ref[...], preferred_element_type=jnp.float32)
    z = z + b_ref[...]
    q_ref[...] = z[:, 0:128]
    k_ref[...] = z[:, 128:256]
    v_ref[...] = z[:, 256:384]
    s_ref[...] = z[:, 384:512]


def _tc_project(xb, wblk, bblk, in_dim):
    out = jax.ShapeDtypeStruct((_NB, _R * CP), jnp.float32)
    return pl.pallas_call(
        _project_body,
        grid=(_NB // _BB,),
        in_specs=[
            pl.BlockSpec((_BB, in_dim), lambda i: (i, 0)),
            pl.BlockSpec((in_dim, 4 * _R * CP), lambda i: (0, 0)),
            pl.BlockSpec((1, 4 * _R * CP), lambda i: (0, 0)),
        ],
        out_specs=[pl.BlockSpec((_BB, _R * CP), lambda i: (i, 0))] * 4,
        out_shape=[out, out, out, out],
    )(xb, wblk, bblk)


def _merge_normalize(acc_ref, mx_ref, skip_ref, sel_ref):
    g0 = jnp.max(mx_ref[0])
    g1 = jnp.max(mx_ref[1])
    g = jnp.maximum(g0, g1)
    a = acc_ref[0] * jnp.exp(g0 - g) + acc_ref[1] * jnp.exp(g1 - g)
    wb = jnp.dot(a, sel_ref[...], preferred_element_type=jnp.float32)
    h = a / (wb + 1e-16) + skip_ref[...]
    h = jnp.maximum(h, 0.0)
    col = lax.broadcasted_iota(jnp.int32, h.shape, 1)
    return jnp.where(col % CP < C, h, 0.0)


def _mid_body(acc_ref, mx_ref, skip_ref, sel_ref, w_ref, b_ref,
              q_ref, k_ref, v_ref, s_ref):
    h = _merge_normalize(acc_ref, mx_ref, skip_ref, sel_ref)
    z = jnp.dot(h, w_ref[...], preferred_element_type=jnp.float32)
    z = z + b_ref[...]
    q_ref[...] = z[:, 0:128]
    k_ref[...] = z[:, 128:256]
    v_ref[...] = z[:, 256:384]
    s_ref[...] = z[:, 384:512]


def _tc_mid(acc, mx, skip, sel, wblk, bblk):
    out = jax.ShapeDtypeStruct((_NB, _R * CP), jnp.float32)
    return pl.pallas_call(
        _mid_body,
        grid=(_NB // _BB,),
        in_specs=[
            pl.BlockSpec((2, _BB, _R * CP), lambda i: (0, i, 0)),
            pl.BlockSpec((2, 16), lambda i: (0, 0)),
            pl.BlockSpec((_BB, _R * CP), lambda i: (i, 0)),
            pl.BlockSpec((_R * CP, _R * CP), lambda i: (0, 0)),
            pl.BlockSpec((_R * CP, 4 * _R * CP), lambda i: (0, 0)),
            pl.BlockSpec((1, 4 * _R * CP), lambda i: (0, 0)),
        ],
        out_specs=[pl.BlockSpec((_BB, _R * CP), lambda i: (i, 0))] * 4,
        out_shape=[out, out, out, out],
    )(acc, mx, skip, sel, wblk, bblk)


def _final_body(acc_ref, mx_ref, skip_ref, sel_ref, wo_ref, bo_ref, y_ref):
    h = _merge_normalize(acc_ref, mx_ref, skip_ref, sel_ref)
    z = jnp.dot(h, wo_ref[...], preferred_element_type=jnp.float32)
    z = z + bo_ref[...]
    y_ref[...] = 1.0 / (1.0 + jnp.exp(-z))


def _tc_final(acc, mx, skip, sel, wo_blk, bo_blk):
    return pl.pallas_call(
        _final_body,
        grid=(_NB // _BB,),
        in_specs=[
            pl.BlockSpec((2, _BB, _R * CP), lambda i: (0, i, 0)),
            pl.BlockSpec((2, 16), lambda i: (0, 0)),
            pl.BlockSpec((_BB, _R * CP), lambda i: (i, 0)),
            pl.BlockSpec((_R * CP, _R * CP), lambda i: (0, 0)),
            pl.BlockSpec((_R * CP, _R * D), lambda i: (0, 0)),
            pl.BlockSpec((1, _R * D), lambda i: (0, 0)),
        ],
        out_specs=pl.BlockSpec((_BB, _R * D), lambda i: (i, 0)),
        out_shape=jax.ShapeDtypeStruct((_NB, _R * D), jnp.float32),
    )(acc, mx, skip, sel, wo_blk, bo_blk)


@functools.lru_cache(maxsize=1)
def _sc_mesh():
    return plsc.VectorSubcoreMesh(
        core_axis_name="c", subcore_axis_name="s", num_cores=2, num_subcores=16)


def _sc_layer_body(q_hbm, k_hbm, v_hbm, ei_hbm,
                   acc_out, mx_out,
                   src_v, dst_v, alpha_vt, qr_all, kr_all, vr_all, msg_all,
                   mxv, mx_v, tmp,
                   acc_spmem, mx_spmem, *sems):
    qrs = [qr_all.at[b] for b in range(_NBUF)]
    krs = [kr_all.at[b] for b in range(_NBUF)]
    vrs = [vr_all.at[b] for b in range(_NBUF)]
    msgs = [msg_all.at[b] for b in range(_NBUF)]
    sems_q = sems[0:_NBUF]
    sems_k = sems[_NBUF:2 * _NBUF]
    sems_v = sems[2 * _NBUF:3 * _NBUF]
    sems_s = sems[3 * _NBUF:4 * _NBUF]

    c = lax.axis_index("c")
    s = lax.axis_index("s")
    wid = c * 16 + s
    base_g = wid * GT
    lane = lax.iota(jnp.int32, 16)
    zero16 = jnp.zeros((16,), jnp.float32)

    pltpu.sync_copy(ei_hbm.at[0, pl.ds(base_g, GT)], src_v)
    pltpu.sync_copy(ei_hbm.at[1, pl.ds(base_g, GT)], dst_v)

    def start_qk(g, b):
        pltpu.make_async_copy(q_hbm.at[dst_v.at[g]], qrs[b], sems_q[b]).start()
        pltpu.make_async_copy(k_hbm.at[src_v.at[g]], krs[b], sems_k[b]).start()

    def wait_qk(g, b):
        pltpu.make_async_copy(q_hbm.at[dst_v.at[g]], qrs[b], sems_q[b]).wait()
        pltpu.make_async_copy(k_hbm.at[src_v.at[g]], krs[b], sems_k[b]).wait()

    def compute_alpha(g, b, mx):
        qr = qrs[b]
        kr = krs[b]
        for sub in range(8):
            idx = lane + (sub * 16)
            acc = jnp.zeros((16,), jnp.float32)
            for ch in range(C):
                chv = jnp.full((16,), ch, jnp.int32)
                qc = plsc.load_gather(qr, [idx, chv])
                kc = plsc.load_gather(kr, [idx, chv])
                acc = acc + qc * kc
            acc = acc * INV_SQRT_C
            alpha_vt[g, pl.ds(sub * 16, 16)] = acc
            mx = jnp.maximum(mx, acc)
        return mx

    for b in range(_NBUF - 1):
        start_qk(b, b)

    def zrow(i, _):
        tmp[i, :] = zero16
        return 0
    lax.fori_loop(0, ROWS_PER_TILE, zrow, 0)

    def zmsg(i, _):
        for b in range(_NBUF):
            msgs[b][i, :] = zero16
        return 0
    lax.fori_loop(0, GROUP, zmsg, 0)

    pltpu.sync_copy(tmp, acc_spmem.at[pl.ds(s * ROWS_PER_TILE, ROWS_PER_TILE)])

    def quad1(it, mx):
        for j in range(_NBUF):
            g = _NBUF * it + j

            @pl.when(g + _NBUF - 1 < GT)
            def _():
                start_qk(g + _NBUF - 1, (j + _NBUF - 1) % _NBUF)

            wait_qk(g, j)
            mx = compute_alpha(g, j, mx)
        return mx

    mx = lax.fori_loop(0, GT // _NBUF, quad1,
                       jnp.full((16,), -1e30, jnp.float32))

    mx_v[...] = mx
    pltpu.sync_copy(mx_v, mx_spmem.at[s])

    def start_v(g, b):
        pltpu.make_async_copy(v_hbm.at[src_v.at[g]], vrs[b], sems_v[b]).start()

    for b in range(_NBUF - 1):
        start_v(b, b)

    plsc.subcore_barrier()

    pltpu.sync_copy(mx_spmem, mxv)
    m = jnp.full((16,), -1e30, jnp.float32)
    for i in range(16):
        m = jnp.maximum(m, mxv[i, :])
    gmax = jnp.max(m)

    @pl.when(s == 0)
    def _():
        mx_v[...] = m
        pltpu.sync_copy(mx_v, mx_out.at[c])

    def wait_v(g, b):
        pltpu.make_async_copy(v_hbm.at[src_v.at[g]], vrs[b], sems_v[b]).wait()

    def wait_scatter(b):
        pltpu.make_async_copy(msgs[b], acc_spmem.at[dst_v.at[0]],
                              sems_s[b]).wait()

    def compute_msg(g, b):
        vr = vrs[b]
        msg = msgs[b]
        for sub in range(8):
            idx = lane + (sub * 16)
            a = alpha_vt[g, pl.ds(sub * 16, 16)]
            ae = jnp.exp(a - gmax)
            gid = (base_g + g) * GROUP + sub * 16 + lane
            ae = jnp.where(gid < E, ae, 0.0)
            for ch in range(C):
                chv = jnp.full((16,), ch, jnp.int32)
                vc = plsc.load_gather(vr, [idx, chv])
                plsc.store_scatter(msg, [idx, chv], vc * ae)
            plsc.store_scatter(msg, [idx, jnp.full((16,), C, jnp.int32)], ae)
        pltpu.async_copy(msg, acc_spmem.at[dst_v.at[g]], sems_s[b], add=True)

    def quad2(it, _):
        for j in range(_NBUF):
            g = _NBUF * it + j

            @pl.when(g + _NBUF - 1 < GT)
            def _():
                start_v(g + _NBUF - 1, (j + _NBUF - 1) % _NBUF)

            wait_v(g, j)

            @pl.when(it > 0)
            def _():
                wait_scatter(j)

            compute_msg(g, j)
        return 0

    lax.fori_loop(0, GT // _NBUF, quad2, 0)
    for b in range(_NBUF):
        wait_scatter(b)
    plsc.subcore_barrier()

    pltpu.sync_copy(acc_spmem.at[pl.ds(s * ROWS_PER_TILE, ROWS_PER_TILE)], tmp)
    pltpu.sync_copy(tmp, acc_out.at[c, pl.ds(s * ROWS_PER_TILE, ROWS_PER_TILE)])


@functools.lru_cache(maxsize=1)
def _sc_layer():
    return pl.kernel(
        _sc_layer_body,
        out_type=[
            jax.ShapeDtypeStruct((2, N, CP), jnp.float32),
            jax.ShapeDtypeStruct((2, 16), jnp.float32),
        ],
        mesh=_sc_mesh(),
        scratch_types=[
            pltpu.VMEM((GT, GROUP), jnp.int32),
            pltpu.VMEM((GT, GROUP), jnp.int32),
            pltpu.VMEM((GT, GROUP), jnp.float32),
            pltpu.VMEM((_NBUF, GROUP, CP), jnp.float32),
            pltpu.VMEM((_NBUF, GROUP, CP), jnp.float32),
            pltpu.VMEM((_NBUF, GROUP, CP), jnp.float32),
            pltpu.VMEM((_NBUF, GROUP, CP), jnp.float32),
            pltpu.VMEM((16, 16), jnp.float32),
            pltpu.VMEM((16,), jnp.float32),
            pltpu.VMEM((ROWS_PER_TILE, CP), jnp.float32),
            pltpu.VMEM_SHARED((N, CP), jnp.float32),
            pltpu.VMEM_SHARED((16, 16), jnp.float32),
        ] + [pltpu.SemaphoreType.DMA] * (4 * _NBUF),
        compiler_params=pltpu.CompilerParams(
            use_tc_tiling_on_sc=False, needs_layout_passes=False),
    )


_EYE8 = _np.eye(_R, dtype=_np.float32)


def _blk_weight(w, in_rows):
    wp = jnp.pad(w, ((0, in_rows - w.shape[0]), (0, CP - w.shape[1])))
    return jnp.kron(_EYE8, wp)


def _pack_weights(wq, bq, wk, bk, wv, bv, ws, bs, in_rows):
    wblk = jnp.concatenate(
        [_blk_weight(w, in_rows) for w in (wq, wk, wv, ws)], axis=1)
    bblk = jnp.concatenate(
        [jnp.tile(jnp.pad(b, (0, CP - C)), _R) for b in (bq, bk, bv, bs)]
    ).reshape(1, 4 * _R * CP)
    return wblk, bblk


def kernel(x, edge_index, Wq1, bq1, Wk1, bk1, Wv1, bv1, Ws1, bs1,
           Wq2, bq2, Wk2, bk2, Wv2, bv2, Ws2, bs2, Wo, bo):
    ei = jnp.pad(edge_index, ((0, 0), (0, EP - E))).reshape(2, W * GT, GROUP)

    wblk1, bblk1 = _pack_weights(Wq1, bq1, Wk1, bk1, Wv1, bv1, Ws1, bs1, D)
    wblk2, bblk2 = _pack_weights(Wq2, bq2, Wk2, bk2, Wv2, bv2, Ws2, bs2, CP)
    wo_blk = jnp.kron(_EYE8, jnp.pad(Wo, ((0, CP - C), (0, 0))))
    bo_blk = jnp.tile(bo, _R).reshape(1, _R * D)
    sel = jnp.asarray(_SEL_NP)

    sc_layer = _sc_layer()

    def t16(a):
        return a.reshape(N, CP)

    xb = x.reshape(_NB, _R * D)
    q1, k1, v1, s1 = _tc_project(xb, wblk1, bblk1, _R * D)
    acc1, mx1 = sc_layer(t16(q1), t16(k1), t16(v1), ei)

    q2, k2, v2, s2 = _tc_mid(acc1.reshape(2, _NB, _R * CP), mx1,
                             s1, sel, wblk2, bblk2)
    acc2, mx2 = sc_layer(t16(q2), t16(k2), t16(v2), ei)

    y = _tc_final(acc2.reshape(2, _NB, _R * CP), mx2, s2, sel, wo_blk, bo_blk)
    return y.reshape(N, D)
